# Initial kernel scaffold; baseline (speedup 1.0000x reference)
import jax
import jax.numpy as jnp
from jax.experimental import pallas as pl


def kernel(x, edge_index, w_mean1, b_mean1, w_var1, b_var1, w_mean2, b_mean2, w_var2, b_var2):
    raise NotImplementedError("write your pallas kernel here")


# speedup vs baseline: 16.7251x; 16.7251x over previous
import functools

import jax
import jax.numpy as jnp
from jax import lax
from jax.experimental import pallas as pl
from jax.experimental.pallas import tpu as pltpu
from jax.experimental.pallas import tpu_sc as plsc

_N = 10000
_E = 320000
_NPAD = 10240
_CH = 80
_EROWS = _E // _CH
_DRPT = _EROWS // 32
_ACH = 125
_AROWS = _E // _ACH
_RPT = _AROWS // 16
_SL = _NPAD // 16
_RB = 1024
_H = 16
_O = 64

_mesh = plsc.VectorSubcoreMesh(core_axis_name="c", subcore_axis_name="s")
_sc_params = pltpu.CompilerParams(use_tc_tiling_on_sc=False)


@functools.partial(
    pl.kernel,
    out_type=jax.ShapeDtypeStruct((2, 1, _NPAD), jnp.float32),
    mesh=_mesh,
    compiler_params=_sc_params,
    scratch_types=[
        pltpu.VMEM((_DRPT, _CH), jnp.int32),
        pltpu.VMEM((_CH,), jnp.float32),
        pltpu.VMEM((_SL,), jnp.float32),
        pltpu.VMEM_SHARED((_NPAD,), jnp.float32),
    ],
)
def _deg_sc(dst_hbm, out_hbm, dst_v, ones_v, zero_v, acc_sh):
    c = lax.axis_index("c")
    s = lax.axis_index("s")

    def fill(i, _):
        zero_v[pl.ds(i * 16, 16)] = jnp.zeros((16,), jnp.float32)
        return _

    lax.fori_loop(0, _SL // 16, fill, None)
    for i in range(_CH // 16):
        ones_v[pl.ds(i * 16, 16)] = jnp.ones((16,), jnp.float32)
    pltpu.sync_copy(zero_v, acc_sh.at[pl.ds(s * _SL, _SL)])
    pltpu.sync_copy(dst_hbm.at[c * 16 + s], dst_v)
    plsc.subcore_barrier()

    def step(j, _):
        pltpu.sync_copy(ones_v, acc_sh.at[dst_v.at[j]], add=True)
        return _

    lax.fori_loop(0, _DRPT, step, None)
    plsc.subcore_barrier()
    pltpu.sync_copy(acc_sh.at[pl.ds(s * _SL, _SL)], out_hbm.at[c, 0, pl.ds(s * _SL, _SL)])


def _make_agg(width, nbuf):
    @functools.partial(
        pl.kernel,
        out_type=jax.ShapeDtypeStruct((2, _NPAD, width), jnp.float32),
        mesh=_mesh,
        compiler_params=_sc_params,
        scratch_types=[
            pltpu.VMEM((_RPT, _ACH), jnp.int32),
            pltpu.VMEM((_RPT, _ACH), jnp.int32),
        ]
        + [pltpu.VMEM((_ACH, width), jnp.float32) for _ in range(nbuf)]
        + [pltpu.VMEM_SHARED((_NPAD, width), jnp.float32)]
        + [pltpu.SemaphoreType.DMA for _ in range(2 * nbuf)],
    )
    def agg(src2_hbm, dst_hbm, msg_hbm, out_hbm, src_v, dst_v, *rest):
        bufs = rest[:nbuf]
        acc_sh = rest[nbuf]
        sgs = rest[nbuf + 1:2 * nbuf + 1]
        sss = rest[2 * nbuf + 1:]
        c = lax.axis_index("c")
        s = lax.axis_index("s")

        def zrow(i, _):
            for t in range(width // 16):
                bufs[0][i, pl.ds(t * 16, 16)] = jnp.zeros((16,), jnp.float32)
            return _

        lax.fori_loop(0, 80, zrow, None)
        zsrc = bufs[0].at[pl.ds(0, 80)]

        def zacc(k, _):
            pltpu.sync_copy(zsrc, acc_sh.at[pl.ds(s * _SL + k * 80, 80)])
            return _

        lax.fori_loop(0, _SL // 80, zacc, None)
        pltpu.sync_copy(src2_hbm.at[c, s], src_v)
        pltpu.sync_copy(dst_hbm.at[s], dst_v)
        plsc.subcore_barrier()

        for b in range(nbuf):
            pltpu.async_copy(msg_hbm.at[src_v.at[b]], bufs[b], sgs[b])

        def outer(g, _):
            for b in range(nbuf):
                m = g * nbuf + b
                pltpu.make_async_copy(msg_hbm.at[src_v.at[m]], bufs[b], sgs[b]).wait()
                pltpu.async_copy(bufs[b], acc_sh.at[dst_v.at[m]], sss[b], add=True)
            for b in range(nbuf):
                m = g * nbuf + b

                @pl.when(m + nbuf < _RPT)
                def _refill():
                    pltpu.make_async_copy(
                        bufs[b], acc_sh.at[dst_v.at[m]], sss[b]
                    ).wait()
                    pltpu.async_copy(msg_hbm.at[src_v.at[m + nbuf]], bufs[b], sgs[b])

            return _

        lax.fori_loop(0, _RPT // nbuf, outer, None)
        for b in range(nbuf):
            m = _RPT - nbuf + b
            pltpu.make_async_copy(bufs[b], acc_sh.at[dst_v.at[m]], sss[b]).wait()
        plsc.subcore_barrier()
        pltpu.sync_copy(
            acc_sh.at[pl.ds(s * _SL, _SL)], out_hbm.at[c, pl.ds(s * _SL, _SL)]
        )

    return agg


_agg_h = _make_agg(_H, 8)
_agg_o = _make_agg(_O, 4)


def _norms(deg_ref):
    deg = deg_ref[:, 0:1] + deg_ref[:, 1:2]
    degc = jnp.maximum(deg, 1.0)
    return lax.rsqrt(degc), 1.0 / degc


def _tc1_body(x_ref, deg_ref, wm_ref, bm_ref, wv_ref, bv_ref, out_ref):
    n1, n2 = _norms(deg_ref)
    xb = x_ref[...]
    mean = jnp.dot(xb, wm_ref[...], preferred_element_type=jnp.float32) + bm_ref[...]
    mean = jnp.maximum(mean, 0.0)
    var = jnp.dot(xb, wv_ref[...], preferred_element_type=jnp.float32) + bv_ref[...]
    var = jnp.maximum(var, 0.0)
    att = jnp.exp(-var)
    out_ref[0] = mean * att * n1
    out_ref[1] = var * att * att * n2


def _tc2_body(agg_ref, deg_ref, wm_ref, bm_ref, wv_ref, bv_ref, out_ref):
    n1, n2 = _norms(deg_ref)
    mean_in = agg_ref[0] * n1
    var_in = agg_ref[1] * n2
    mean = (
        jnp.dot(mean_in, wm_ref[...], preferred_element_type=jnp.float32)
        + bm_ref[...]
    )
    var = (
        jnp.dot(var_in, wv_ref[...], preferred_element_type=jnp.float32)
        + bv_ref[...]
    )
    var = jnp.maximum(var, 0.0)
    att = jnp.exp(-var)
    out_ref[0] = mean * att * n1
    out_ref[1] = var * att * att * n2


def _tcf_body(agg_ref, deg_ref, eps_ref, out_ref):
    n1, n2 = _norms(deg_ref)
    mean = agg_ref[0] * n1
    var = agg_ref[1] * n2
    out_ref[...] = eps_ref[...] * jnp.sqrt(var + 1e-8) + mean


def _tc1(x, deg2t, wm, bm, wv, bv):
    return pl.pallas_call(
        _tc1_body,
        grid=(_NPAD // _RB,),
        in_specs=[
            pl.BlockSpec((_RB, 128), lambda i: (i, 0)),
            pl.BlockSpec((_RB, 2), lambda i: (i, 0)),
            pl.BlockSpec((128, _H), lambda i: (0, 0)),
            pl.BlockSpec((1, _H), lambda i: (0, 0)),
            pl.BlockSpec((128, _H), lambda i: (0, 0)),
            pl.BlockSpec((1, _H), lambda i: (0, 0)),
        ],
        out_specs=pl.BlockSpec((2, _RB, _H), lambda i: (0, i, 0)),
        out_shape=jax.ShapeDtypeStruct((2, _NPAD, _H), jnp.float32),
    )(x, deg2t, wm, bm.reshape(1, _H), wv, bv.reshape(1, _H))


def _tc2(agg1, deg2t, wm, bm, wv, bv):
    return pl.pallas_call(
        _tc2_body,
        grid=(_NPAD // _RB,),
        in_specs=[
            pl.BlockSpec((2, _RB, _H), lambda i: (0, i, 0)),
            pl.BlockSpec((_RB, 2), lambda i: (i, 0)),
            pl.BlockSpec((_H, _O), lambda i: (0, 0)),
            pl.BlockSpec((1, _O), lambda i: (0, 0)),
            pl.BlockSpec((_H, _O), lambda i: (0, 0)),
            pl.BlockSpec((1, _O), lambda i: (0, 0)),
        ],
        out_specs=pl.BlockSpec((2, _RB, _O), lambda i: (0, i, 0)),
        out_shape=jax.ShapeDtypeStruct((2, _NPAD, _O), jnp.float32),
    )(agg1, deg2t, wm, bm.reshape(1, _O), wv, bv.reshape(1, _O))


def _tcf(agg2, deg2t, eps):
    return pl.pallas_call(
        _tcf_body,
        grid=(_NPAD // _RB,),
        in_specs=[
            pl.BlockSpec((2, _RB, _O), lambda i: (0, i, 0)),
            pl.BlockSpec((_RB, 2), lambda i: (i, 0)),
            pl.BlockSpec((_RB, _O), lambda i: (i, 0)),
        ],
        out_specs=pl.BlockSpec((_RB, _O), lambda i: (i, 0)),
        out_shape=jax.ShapeDtypeStruct((_NPAD, _O), jnp.float32),
    )(agg2, deg2t, eps)


def kernel(x, edge_index, w_mean1, b_mean1, w_var1, b_var1,
           w_mean2, b_mean2, w_var2, b_var2):
    src = edge_index[0]
    dst = edge_index[1]
    src4 = jnp.stack([src, src + _NPAD]).reshape(2, 16, _RPT, _ACH)
    dst_deg = dst.reshape(32, _DRPT, _CH)
    dst_agg = dst.reshape(16, _RPT, _ACH)

    deg2 = _deg_sc(dst_deg).reshape(2, _NPAD)
    deg2t = deg2.T

    msg1 = _tc1(x, deg2t, w_mean1, b_mean1, w_var1, b_var1)
    agg1 = _agg_h(src4, dst_agg, msg1.reshape(2 * _NPAD, _H))
    msg2 = _tc2(agg1, deg2t, w_mean2, b_mean2, w_var2, b_var2)
    agg2 = _agg_o(src4, dst_agg, msg2.reshape(2 * _NPAD, _O))

    eps = jax.random.normal(jax.random.key(42), (_N, _O), jnp.float32)
    z = _tcf(agg2, deg2t, eps)
    return z[:_N]
